# baseline (speedup 1.0000x reference)
import functools

import jax
import jax.numpy as jnp
from jax import lax
from jax.experimental import pallas as pl
from jax.experimental.pallas import tpu as pltpu
from jax.experimental.pallas import tpu_sc as plsc
import numpy as np

B, E_DIM, L = 8, 384, 576
N_E, H_DIM = 8192, 64
BETA = 0.25
EPS = float(np.finfo(np.float32).eps)
T = B * L

KB = 512
NKB = N_E // KB

NW = 32
RPW = T // NW
RH = RPW // 2


def _cb_norm_body(cb_ref, out_ref):
    cb = cb_ref[...]
    cms = jnp.mean(cb * cb, axis=1, keepdims=True)
    cbn = cb * lax.rsqrt(cms + EPS)
    out_ref[...] = jnp.concatenate(
        [cbn, jnp.zeros_like(cbn)], axis=1)


_cb_norm = pl.pallas_call(
    _cb_norm_body,
    grid=(8,),
    in_specs=[pl.BlockSpec((N_E // 8, H_DIM), lambda i: (i, 0))],
    out_specs=pl.BlockSpec((N_E // 8, 2 * H_DIM), lambda i: (i, 0)),
    out_shape=jax.ShapeDtypeStruct((N_E, 2 * H_DIM), jnp.float32),
)


def _sim_argmax_body(z_ref, w_in_ref, b_in_ref, cb_ref,
                     inds_ref, maxv_ref, zn2_ref,
                     zn_s, mx_s, ix_s):
    k = pl.program_id(1)

    @pl.when(k == 0)
    def _():
        zb = z_ref[0]
        zp = jnp.dot(w_in_ref[...], zb,
                     preferred_element_type=jnp.float32) + b_in_ref[...]
        ms = jnp.mean(zp * zp, axis=0, keepdims=True)
        zn = zp * lax.rsqrt(ms + EPS)
        zn_s[...] = zn
        zn2_ref[0] = jnp.sum(zn * zn, axis=0, keepdims=True)
        mx_s[...] = jnp.full((1, L), -jnp.inf, jnp.float32)
        ix_s[...] = jnp.zeros((1, L), jnp.int32)

    cb = cb_ref[...]
    cms = jnp.mean(cb * cb, axis=1, keepdims=True)
    cbn = cb * lax.rsqrt(cms + EPS)
    s = jnp.dot(cbn, zn_s[...], preferred_element_type=jnp.float32)
    m = jnp.max(s, axis=0, keepdims=True)
    rows = lax.broadcasted_iota(jnp.int32, s.shape, 0)
    li = jnp.min(jnp.where(s == m, rows, KB), axis=0, keepdims=True)
    upd = m > mx_s[...]
    ix_s[...] = jnp.where(upd, li + k * KB, ix_s[...])
    mx_s[...] = jnp.where(upd, m, mx_s[...])

    @pl.when(k == pl.num_programs(1) - 1)
    def _():
        inds_ref[0] = ix_s[...]
        maxv_ref[0] = mx_s[...]


_sim_argmax = pl.pallas_call(
    _sim_argmax_body,
    grid=(B, NKB),
    in_specs=[
        pl.BlockSpec((1, E_DIM, L), lambda b, k: (b, 0, 0)),
        pl.BlockSpec((H_DIM, E_DIM), lambda b, k: (0, 0)),
        pl.BlockSpec((H_DIM, 1), lambda b, k: (0, 0)),
        pl.BlockSpec((KB, H_DIM), lambda b, k: (k, 0)),
    ],
    out_specs=[
        pl.BlockSpec((1, 1, L), lambda b, k: (b, 0, 0)),
        pl.BlockSpec((1, 1, L), lambda b, k: (b, 0, 0)),
        pl.BlockSpec((1, 1, L), lambda b, k: (b, 0, 0)),
    ],
    out_shape=[
        jax.ShapeDtypeStruct((B, 1, L), jnp.int32),
        jax.ShapeDtypeStruct((B, 1, L), jnp.float32),
        jax.ShapeDtypeStruct((B, 1, L), jnp.float32),
    ],
    scratch_shapes=[
        pltpu.VMEM((H_DIM, L), jnp.float32),
        pltpu.VMEM((1, L), jnp.float32),
        pltpu.VMEM((1, L), jnp.int32),
    ],
)


@functools.cache
def _make_sc_gather():
    mesh = plsc.VectorSubcoreMesh(core_axis_name="c", subcore_axis_name="s")

    @functools.partial(
        pl.kernel, mesh=mesh,
        out_type=jax.ShapeDtypeStruct((T, 2 * H_DIM), jnp.float32),
        scratch_types=[
            pltpu.VMEM((RH,), jnp.int32),
            pltpu.VMEM((RH,), jnp.int32),
            pltpu.VMEM((RPW, 2 * H_DIM), jnp.float32),
            pltpu.SemaphoreType.DMA,
        ],
    )
    def gather_k(table_hbm, idx_hbm, out_hbm, idx_a, idx_b, rows_v, sem):
        wid = lax.axis_index("s") * 2 + lax.axis_index("c")
        base = wid * RPW
        pltpu.sync_copy(idx_hbm.at[pl.ds(base, RH)], idx_a)
        pltpu.sync_copy(idx_hbm.at[pl.ds(base + RH, RH)], idx_b)
        c1 = pltpu.async_copy(table_hbm.at[idx_a], rows_v.at[pl.ds(0, RH)], sem)
        c2 = pltpu.async_copy(table_hbm.at[idx_b], rows_v.at[pl.ds(RH, RH)], sem)
        c1.wait()
        c2.wait()
        pltpu.sync_copy(rows_v, out_hbm.at[pl.ds(base, RPW)])

    return gather_k


def _proj_out_body(zq_ref, maxv_ref, zn2_ref, w_out_ref, b_out_ref,
                   out_ref, loss_ref, acc):
    b = pl.program_id(0)
    zq = zq_ref[0][:, :H_DIM]
    o = lax.dot_general(w_out_ref[...], zq, (((1,), (1,)), ((), ())),
                        preferred_element_type=jnp.float32)
    out_ref[0] = o + b_out_ref[...]
    part = (jnp.sum(zq * zq) - 2.0 * jnp.sum(maxv_ref[0])
            + jnp.sum(zn2_ref[0]))

    @pl.when(b == 0)
    def _():
        acc[0, 0] = 0.0

    acc[0, 0] += part

    @pl.when(b == pl.num_programs(0) - 1)
    def _():
        loss_ref[0, 0] = acc[0, 0] * ((1.0 + BETA) / float(T * H_DIM))


_proj_out = pl.pallas_call(
    _proj_out_body,
    grid=(B,),
    in_specs=[
        pl.BlockSpec((1, L, 2 * H_DIM), lambda b: (b, 0, 0)),
        pl.BlockSpec((1, 1, L), lambda b: (b, 0, 0)),
        pl.BlockSpec((1, 1, L), lambda b: (b, 0, 0)),
        pl.BlockSpec((E_DIM, H_DIM), lambda b: (0, 0)),
        pl.BlockSpec((E_DIM, 1), lambda b: (0, 0)),
    ],
    out_specs=[
        pl.BlockSpec((1, E_DIM, L), lambda b: (b, 0, 0)),
        pl.BlockSpec(memory_space=pltpu.SMEM),
    ],
    out_shape=[
        jax.ShapeDtypeStruct((B, E_DIM, L), jnp.float32),
        jax.ShapeDtypeStruct((1, 1), jnp.float32),
    ],
    scratch_shapes=[
        pltpu.SMEM((1, 1), jnp.float32),
    ],
)


def kernel(z, W_in, b_in, codebook, W_out, b_out):
    cb_n_pad = _cb_norm(codebook)
    inds3, maxv, zn2 = _sim_argmax(z, W_in, b_in.reshape(H_DIM, 1), codebook)
    inds = inds3.reshape(B, L)
    zq_rows = _make_sc_gather()(cb_n_pad, inds.reshape(T))
    out, loss = _proj_out(zq_rows.reshape(B, L, 2 * H_DIM), maxv, zn2,
                          W_out, b_out.reshape(E_DIM, 1))
    return out, inds, loss.reshape(())


# speedup vs baseline: 1.4210x; 1.4210x over previous
import functools

import jax
import jax.numpy as jnp
from jax import lax
from jax.experimental import pallas as pl
from jax.experimental.pallas import tpu as pltpu
from jax.experimental.pallas import tpu_sc as plsc
import numpy as np

B, E_DIM, L = 8, 384, 576
N_E, H_DIM = 8192, 64
BETA = 0.25
EPS = float(np.finfo(np.float32).eps)
T = B * L

KB = 1024
NKB = N_E // KB
SUB = 256

NW = 32
RPW = T // NW
RH = RPW // 2


def _sim_argmax_body(z_ref, w_in_ref, b_in_ref, cb_ref,
                     inds_ref, loss_ref, cbt_ref,
                     zn_s, cbn_s, mx_s, ix_s, acc):
    k = pl.program_id(0)
    b = pl.program_id(1)

    @pl.when(k == 0)
    def _():
        zb = z_ref[0]
        zp = jnp.dot(w_in_ref[...], zb,
                     preferred_element_type=jnp.float32) + b_in_ref[...]
        ms = jnp.mean(zp * zp, axis=0, keepdims=True)
        zn = zp * lax.rsqrt(ms + EPS)
        zn_s[b] = zn
        zn2 = jnp.float32(H_DIM) * ms / (ms + EPS)

        @pl.when(b == 0)
        def _():
            acc[0, 0] = 0.0
        acc[0, 0] += jnp.sum(zn2)
        mx_s[b] = jnp.full((1, L), -jnp.inf, jnp.float32)
        ix_s[b] = jnp.zeros((1, L), jnp.int32)

    @pl.when(b == 0)
    def _():
        cb = cb_ref[...]
        cms = jnp.mean(cb * cb, axis=1, keepdims=True)
        cbn = cb * lax.rsqrt(cms + EPS)
        cbn_s[...] = cbn
        cbt_ref[...] = jnp.concatenate(
            [cbn, jnp.zeros_like(cbn)], axis=1)

    zn = zn_s[b]
    m = mx_s[b]
    li = ix_s[b]
    for j in range(KB // SUB):
        sj = jnp.dot(cbn_s[pl.ds(j * SUB, SUB), :], zn,
                     preferred_element_type=jnp.float32)
        mj = jnp.max(sj, axis=0, keepdims=True)
        rows = lax.broadcasted_iota(jnp.int32, sj.shape, 0)
        lj = jnp.min(jnp.where(sj == mj, rows, SUB), axis=0,
                     keepdims=True) + (k * KB + j * SUB)
        upd = mj > m
        li = jnp.where(upd, lj, li)
        m = jnp.where(upd, mj, m)
    ix_s[b] = li
    mx_s[b] = m

    @pl.when(k == pl.num_programs(0) - 1)
    def _():
        inds_ref[0] = ix_s[b]
        acc[0, 0] += -2.0 * jnp.sum(mx_s[b]) + jnp.float32(H_DIM * L)

        @pl.when(b == pl.num_programs(1) - 1)
        def _():
            loss_ref[0, 0] = acc[0, 0] * ((1.0 + BETA) / float(T * H_DIM))


_sim_argmax = pl.pallas_call(
    _sim_argmax_body,
    grid=(NKB, B),
    in_specs=[
        pl.BlockSpec((1, E_DIM, L),
                     lambda k, b: (jnp.where(k == 0, b, B - 1), 0, 0)),
        pl.BlockSpec((H_DIM, E_DIM), lambda k, b: (0, 0)),
        pl.BlockSpec((H_DIM, 1), lambda k, b: (0, 0)),
        pl.BlockSpec((KB, H_DIM), lambda k, b: (k, 0)),
    ],
    out_specs=[
        pl.BlockSpec((1, 1, L), lambda k, b: (b, 0, 0)),
        pl.BlockSpec(memory_space=pltpu.SMEM),
        pl.BlockSpec((KB, 2 * H_DIM), lambda k, b: (k, 0)),
    ],
    out_shape=[
        jax.ShapeDtypeStruct((B, 1, L), jnp.int32),
        jax.ShapeDtypeStruct((1, 1), jnp.float32),
        jax.ShapeDtypeStruct((N_E, 2 * H_DIM), jnp.float32),
    ],
    scratch_shapes=[
        pltpu.VMEM((B, H_DIM, L), jnp.float32),
        pltpu.VMEM((KB, H_DIM), jnp.float32),
        pltpu.VMEM((B, 1, L), jnp.float32),
        pltpu.VMEM((B, 1, L), jnp.int32),
        pltpu.SMEM((1, 1), jnp.float32),
    ],
)


@functools.cache
def _make_sc_gather():
    mesh = plsc.VectorSubcoreMesh(core_axis_name="c", subcore_axis_name="s")

    @functools.partial(
        pl.kernel, mesh=mesh,
        out_type=jax.ShapeDtypeStruct((T, 2 * H_DIM), jnp.float32),
        scratch_types=[
            pltpu.VMEM((RH,), jnp.int32),
            pltpu.VMEM((RH,), jnp.int32),
            pltpu.VMEM((RPW, 2 * H_DIM), jnp.float32),
            pltpu.SemaphoreType.DMA,
        ],
    )
    def gather_k(table_hbm, idx_hbm, out_hbm, idx_a, idx_b, rows_v, sem):
        wid = lax.axis_index("s") * 2 + lax.axis_index("c")
        base = wid * RPW
        pltpu.sync_copy(idx_hbm.at[pl.ds(base, RH)], idx_a)
        pltpu.sync_copy(idx_hbm.at[pl.ds(base + RH, RH)], idx_b)
        c1 = pltpu.async_copy(table_hbm.at[idx_a], rows_v.at[pl.ds(0, RH)], sem)
        c2 = pltpu.async_copy(table_hbm.at[idx_b], rows_v.at[pl.ds(RH, RH)], sem)
        c1.wait()
        c2.wait()
        pltpu.sync_copy(rows_v, out_hbm.at[pl.ds(base, RPW)])

    return gather_k


def _proj_out_body(zq_ref, w_out_ref, b_out_ref, out_ref):
    zq = zq_ref[0][:, :H_DIM]
    o = lax.dot_general(w_out_ref[...], zq, (((1,), (1,)), ((), ())),
                        preferred_element_type=jnp.float32)
    out_ref[0] = o + b_out_ref[...]


_proj_out = pl.pallas_call(
    _proj_out_body,
    grid=(B,),
    in_specs=[
        pl.BlockSpec((1, L, 2 * H_DIM), lambda b: (b, 0, 0)),
        pl.BlockSpec((E_DIM, H_DIM), lambda b: (0, 0)),
        pl.BlockSpec((E_DIM, 1), lambda b: (0, 0)),
    ],
    out_specs=pl.BlockSpec((1, E_DIM, L), lambda b: (b, 0, 0)),
    out_shape=jax.ShapeDtypeStruct((B, E_DIM, L), jnp.float32),
)


def kernel(z, W_in, b_in, codebook, W_out, b_out):
    inds3, loss, cb_n_pad = _sim_argmax(z, W_in, b_in.reshape(H_DIM, 1),
                                        codebook)
    inds = inds3.reshape(B, L)
    zq_rows = _make_sc_gather()(cb_n_pad, inds.reshape(T))
    out = _proj_out(zq_rows.reshape(B, L, 2 * H_DIM), W_out,
                    b_out.reshape(E_DIM, 1))
    return out, inds, loss.reshape(())
